# baseline (speedup 1.0000x reference)
import functools

import jax
import jax.numpy as jnp
from jax import lax
from jax.experimental import pallas as pl
from jax.experimental.pallas import tpu as pltpu
from jax.experimental.pallas import tpu_sc as plsc

_BATCH = 16384
_SEQ = 50
_DIM = 32

_NC = 2
_NS = 16
_NW = _NC * _NS
_BPW = _BATCH // _NW
_NBUF = 4


def _pool_body(x_hbm, m_hbm, table_hbm, out_hbm,
               idx_v, mask_v, rows0, rows1, rows2, rows3, pooled_v,
               sem0, sem1, sem2, sem3):
    bufs = (rows0, rows1, rows2, rows3)
    sems = (sem0, sem1, sem2, sem3)
    wid = lax.axis_index("s") * _NC + lax.axis_index("c")

    pltpu.sync_copy(x_hbm.at[pl.ds(wid * _BPW, _BPW)], idx_v)
    pltpu.sync_copy(m_hbm.at[pl.ds(wid * _BPW, _BPW)], mask_v)

    def start(r, buf, sem):
        pltpu.async_copy(table_hbm.at[idx_v.at[r]], buf, sem)

    def wait(r, buf, sem):
        pltpu.make_async_copy(table_hbm.at[idx_v.at[r]], buf, sem).wait()

    def compute(r, buf):
        mv = [mask_v[r, pl.ds(0, 16)],
              mask_v[r, pl.ds(16, 16)],
              mask_v[r, pl.ds(32, 16)],
              mask_v[r, pl.ds(34, 16)]]
        acc = [jnp.zeros((16,), jnp.float32) for _ in range(4)]
        for l in range(_SEQ):
            if l < 48:
                m = mv[l // 16][l % 16]
            else:
                m = mv[3][l - 34]
            acc[l % 2] = acc[l % 2] + m * buf[l, 0:16]
            acc[2 + l % 2] = acc[2 + l % 2] + m * buf[l, 16:32]
        pooled_v[r, 0:16] = acc[0] + acc[1]
        pooled_v[r, 16:32] = acc[2] + acc[3]

    for k in range(_NBUF):
        start(k, bufs[k], sems[k])

    def step(i, _):
        jj = _NBUF * i
        for r in range(_NBUF):
            j = jj + r
            wait(j, bufs[r], sems[r])
            compute(j, bufs[r])

            @pl.when(j + _NBUF < _BPW)
            def _():
                start(j + _NBUF, bufs[r], sems[r])

        return _

    lax.fori_loop(0, _BPW // _NBUF, step, None)

    pltpu.sync_copy(pooled_v, out_hbm.at[pl.ds(wid * _BPW, _BPW)])


@functools.partial(
    pl.kernel,
    out_type=jax.ShapeDtypeStruct((_BATCH, _DIM), jnp.float32),
    mesh=plsc.VectorSubcoreMesh(core_axis_name="c", subcore_axis_name="s"),
    compiler_params=pltpu.CompilerParams(use_tc_tiling_on_sc=False),
    scratch_types=[
        pltpu.VMEM((_BPW, _SEQ), jnp.int32),
        pltpu.VMEM((_BPW, _SEQ), jnp.float32),
        pltpu.VMEM((_SEQ, _DIM), jnp.float32),
        pltpu.VMEM((_SEQ, _DIM), jnp.float32),
        pltpu.VMEM((_SEQ, _DIM), jnp.float32),
        pltpu.VMEM((_SEQ, _DIM), jnp.float32),
        pltpu.VMEM((_BPW, _DIM), jnp.float32),
        pltpu.SemaphoreType.DMA,
        pltpu.SemaphoreType.DMA,
        pltpu.SemaphoreType.DMA,
        pltpu.SemaphoreType.DMA,
    ],
)
def _pool(x_hbm, m_hbm, table_hbm, out_hbm,
          idx_v, mask_v, rows0, rows1, rows2, rows3, pooled_v,
          sem0, sem1, sem2, sem3):
    _pool_body(x_hbm, m_hbm, table_hbm, out_hbm,
               idx_v, mask_v, rows0, rows1, rows2, rows3, pooled_v,
               sem0, sem1, sem2, sem3)


_MM_BLK = 2048


def _mm_body(s_ref, wt_ref, b_ref, o_ref):
    acc = jnp.dot(s_ref[...], wt_ref[...], preferred_element_type=jnp.float32)
    o_ref[...] = acc * (1.0 / _SEQ) + b_ref[...]


def _linear(s, wt, b2):
    return pl.pallas_call(
        _mm_body,
        out_shape=jax.ShapeDtypeStruct((_BATCH, _DIM), jnp.float32),
        grid=(_BATCH // _MM_BLK,),
        in_specs=[
            pl.BlockSpec((_MM_BLK, _DIM), lambda i: (i, 0)),
            pl.BlockSpec((_DIM, _DIM), lambda i: (0, 0)),
            pl.BlockSpec((1, _DIM), lambda i: (0, 0)),
        ],
        out_specs=pl.BlockSpec((_MM_BLK, _DIM), lambda i: (i, 0)),
    )(s, wt, b2)


@jax.jit
def kernel(x, mask, table, W, b):
    pooled = _pool(x.astype(jnp.int32), mask, table)
    return _linear(pooled, W.T, b.reshape(1, _DIM))


# speedup vs baseline: 1.0007x; 1.0007x over previous
import functools

import jax
import jax.numpy as jnp
from jax import lax
from jax.experimental import pallas as pl
from jax.experimental.pallas import tpu as pltpu
from jax.experimental.pallas import tpu_sc as plsc

_BATCH = 16384
_SEQ = 50
_DIM = 32

_NC = 2
_NS = 16
_NW = _NC * _NS
_BPW = _BATCH // _NW
_NBUF = 4


def _pool_body(x_hbm, m_hbm, table_hbm, out_hbm,
               idx_v, mask_v, rows0, rows1, rows2, rows3, pooled_v,
               sem0, sem1, sem2, sem3):
    bufs = (rows0, rows1, rows2, rows3)
    sems = (sem0, sem1, sem2, sem3)
    wid = lax.axis_index("s") * _NC + lax.axis_index("c")

    pltpu.sync_copy(x_hbm.at[pl.ds(wid * _BPW, _BPW)], idx_v)
    pltpu.sync_copy(m_hbm.at[pl.ds(wid * _BPW, _BPW)], mask_v)

    def start(r, buf, sem):
        pltpu.async_copy(table_hbm.at[idx_v.at[r]], buf, sem)

    def wait(r, buf, sem):
        pltpu.make_async_copy(table_hbm.at[idx_v.at[r]], buf, sem).wait()

    def compute(r, buf):
        mv = [mask_v[r, pl.ds(0, 16)],
              mask_v[r, pl.ds(16, 16)],
              mask_v[r, pl.ds(32, 16)],
              mask_v[r, pl.ds(34, 16)]]
        acc = [jnp.zeros((16,), jnp.float32) for _ in range(4)]
        for l in range(_SEQ):
            if l < 48:
                m = mv[l // 16][l % 16]
            else:
                m = mv[3][l - 34]
            acc[l % 2] = acc[l % 2] + m * buf[l, 0:16]
            acc[2 + l % 2] = acc[2 + l % 2] + m * buf[l, 16:32]
        pooled_v[r, 0:16] = acc[0] + acc[1]
        pooled_v[r, 16:32] = acc[2] + acc[3]

    for k in range(_NBUF):
        start(k, bufs[k], sems[k])

    def step(i, _):
        jj = _NBUF * i
        for r in range(_NBUF):
            j = jj + r
            wait(j, bufs[r], sems[r])
            compute(j, bufs[r])

            @pl.when(j + _NBUF < _BPW)
            def _():
                start(j + _NBUF, bufs[r], sems[r])

        return _

    lax.fori_loop(0, _BPW // _NBUF, step, None)

    pltpu.sync_copy(pooled_v, out_hbm.at[pl.ds(wid * _BPW, _BPW)])


@functools.partial(
    pl.kernel,
    out_type=jax.ShapeDtypeStruct((_BATCH, _DIM), jnp.float32),
    mesh=plsc.VectorSubcoreMesh(core_axis_name="c", subcore_axis_name="s"),
    compiler_params=pltpu.CompilerParams(use_tc_tiling_on_sc=False),
    scratch_types=[
        pltpu.VMEM((_BPW, _SEQ), jnp.int32),
        pltpu.VMEM((_BPW, _SEQ), jnp.float32),
        pltpu.VMEM((_SEQ, _DIM), jnp.float32),
        pltpu.VMEM((_SEQ, _DIM), jnp.float32),
        pltpu.VMEM((_SEQ, _DIM), jnp.float32),
        pltpu.VMEM((_SEQ, _DIM), jnp.float32),
        pltpu.VMEM((_BPW, _DIM), jnp.float32),
        pltpu.SemaphoreType.DMA,
        pltpu.SemaphoreType.DMA,
        pltpu.SemaphoreType.DMA,
        pltpu.SemaphoreType.DMA,
    ],
)
def _pool(x_hbm, m_hbm, table_hbm, out_hbm,
          idx_v, mask_v, rows0, rows1, rows2, rows3, pooled_v,
          sem0, sem1, sem2, sem3):
    _pool_body(x_hbm, m_hbm, table_hbm, out_hbm,
               idx_v, mask_v, rows0, rows1, rows2, rows3, pooled_v,
               sem0, sem1, sem2, sem3)


_MM_BLK = 2048

_VOCAB = 1000000
_TR_C = 2048
_TR_ROWS = _TR_C * _DIM // 128
_FLAT_ROWS = _VOCAB * _DIM // 128


def _tr_body(t_ref, o_ref):
    blk = t_ref[...]
    o_ref[...] = blk.T.reshape(_TR_ROWS, 128)


def _relayout(tt):
    grid = (_VOCAB + _TR_C - 1) // _TR_C
    return pl.pallas_call(
        _tr_body,
        out_shape=jax.ShapeDtypeStruct((_FLAT_ROWS, 128), jnp.float32),
        grid=(grid,),
        in_specs=[pl.BlockSpec((_DIM, _TR_C), lambda j: (0, j))],
        out_specs=pl.BlockSpec((_TR_ROWS, 128), lambda j: (j, 0)),
    )(tt)


def _mm_body(s_ref, wt_ref, b_ref, o_ref):
    acc = jnp.dot(s_ref[...], wt_ref[...], preferred_element_type=jnp.float32)
    o_ref[...] = acc * (1.0 / _SEQ) + b_ref[...]


def _linear(s, wt, b2):
    return pl.pallas_call(
        _mm_body,
        out_shape=jax.ShapeDtypeStruct((_BATCH, _DIM), jnp.float32),
        grid=(_BATCH // _MM_BLK,),
        in_specs=[
            pl.BlockSpec((_MM_BLK, _DIM), lambda i: (i, 0)),
            pl.BlockSpec((_DIM, _DIM), lambda i: (0, 0)),
            pl.BlockSpec((1, _DIM), lambda i: (0, 0)),
        ],
        out_specs=pl.BlockSpec((_MM_BLK, _DIM), lambda i: (i, 0)),
    )(s, wt, b2)


@jax.jit
def kernel(x, mask, table, W, b):
    pooled = _pool(x.astype(jnp.int32), mask, table)
    return _linear(pooled, W.T, b.reshape(1, _DIM))


# speedup vs baseline: 1.0628x; 1.0620x over previous
import functools

import jax
import jax.numpy as jnp
from jax import lax
from jax.experimental import pallas as pl
from jax.experimental.pallas import tpu as pltpu
from jax.experimental.pallas import tpu_sc as plsc

_BATCH = 16384
_SEQ = 50
_DIM = 32

_NC = 2
_NS = 16
_NW = _NC * _NS
_BPW = _BATCH // _NW
_TOK = 2 * _SEQ
_GPW = _BPW // 2
_NBUF = 4


def _pool_body(x_hbm, m_hbm, table_hbm, out_hbm,
               idx_v, mask_v, rows0, rows1, rows2, rows3, pooled_v,
               sem0, sem1, sem2, sem3):
    bufs = (rows0, rows1, rows2, rows3)
    sems = (sem0, sem1, sem2, sem3)
    wid = lax.axis_index("s") * _NC + lax.axis_index("c")

    pltpu.sync_copy(x_hbm.at[pl.ds(wid * _GPW, _GPW)], idx_v)
    pltpu.sync_copy(m_hbm.at[pl.ds(wid * _GPW, _GPW)], mask_v)

    def start(r, buf, sem):
        pltpu.async_copy(table_hbm.at[idx_v.at[r]], buf, sem)

    def wait(r, buf, sem):
        pltpu.make_async_copy(table_hbm.at[idx_v.at[r]], buf, sem).wait()

    def compute(r, buf):
        for h in range(2):
            o = h * _SEQ
            mv = [mask_v[r, pl.ds(o + 0, 16)],
                  mask_v[r, pl.ds(o + 16, 16)],
                  mask_v[r, pl.ds(o + 32, 16)],
                  mask_v[r, pl.ds(o + 34, 16)]]
            acc = [jnp.zeros((16,), jnp.float32) for _ in range(4)]
            for l in range(_SEQ):
                if l < 48:
                    m = mv[l // 16][l % 16]
                else:
                    m = mv[3][l - 34]
                acc[l % 2] = acc[l % 2] + m * buf[o + l, 0:16]
                acc[2 + l % 2] = acc[2 + l % 2] + m * buf[o + l, 16:32]
            pooled_v[2 * r + h, 0:16] = acc[0] + acc[1]
            pooled_v[2 * r + h, 16:32] = acc[2] + acc[3]

    for k in range(_NBUF):
        start(k, bufs[k], sems[k])

    def step(i, _):
        jj = _NBUF * i
        for r in range(_NBUF):
            j = jj + r
            wait(j, bufs[r], sems[r])
            compute(j, bufs[r])

            @pl.when(j + _NBUF < _GPW)
            def _():
                start(j + _NBUF, bufs[r], sems[r])

        return _

    lax.fori_loop(0, _GPW // _NBUF, step, None)

    pltpu.sync_copy(pooled_v, out_hbm.at[pl.ds(wid * _BPW, _BPW)])


@functools.partial(
    pl.kernel,
    out_type=jax.ShapeDtypeStruct((_BATCH, _DIM), jnp.float32),
    mesh=plsc.VectorSubcoreMesh(core_axis_name="c", subcore_axis_name="s"),
    compiler_params=pltpu.CompilerParams(use_tc_tiling_on_sc=False),
    scratch_types=[
        pltpu.VMEM((_GPW, _TOK), jnp.int32),
        pltpu.VMEM((_GPW, _TOK), jnp.float32),
        pltpu.VMEM((_TOK, _DIM), jnp.float32),
        pltpu.VMEM((_TOK, _DIM), jnp.float32),
        pltpu.VMEM((_TOK, _DIM), jnp.float32),
        pltpu.VMEM((_TOK, _DIM), jnp.float32),
        pltpu.VMEM((_BPW, _DIM), jnp.float32),
        pltpu.SemaphoreType.DMA,
        pltpu.SemaphoreType.DMA,
        pltpu.SemaphoreType.DMA,
        pltpu.SemaphoreType.DMA,
    ],
)
def _pool(x_hbm, m_hbm, table_hbm, out_hbm,
          idx_v, mask_v, rows0, rows1, rows2, rows3, pooled_v,
          sem0, sem1, sem2, sem3):
    _pool_body(x_hbm, m_hbm, table_hbm, out_hbm,
               idx_v, mask_v, rows0, rows1, rows2, rows3, pooled_v,
               sem0, sem1, sem2, sem3)


_MM_BLK = 2048


def _mm_body(s_ref, wt_ref, b_ref, o_ref):
    acc = jnp.dot(s_ref[...], wt_ref[...], preferred_element_type=jnp.float32)
    o_ref[...] = acc * (1.0 / _SEQ) + b_ref[...]


def _linear(s, wt, b2):
    return pl.pallas_call(
        _mm_body,
        out_shape=jax.ShapeDtypeStruct((_BATCH, _DIM), jnp.float32),
        grid=(_BATCH // _MM_BLK,),
        in_specs=[
            pl.BlockSpec((_MM_BLK, _DIM), lambda i: (i, 0)),
            pl.BlockSpec((_DIM, _DIM), lambda i: (0, 0)),
            pl.BlockSpec((1, _DIM), lambda i: (0, 0)),
        ],
        out_specs=pl.BlockSpec((_MM_BLK, _DIM), lambda i: (i, 0)),
    )(s, wt, b2)


@jax.jit
def kernel(x, mask, table, W, b):
    x2 = x.astype(jnp.int32).reshape(_BATCH // 2, _TOK)
    m2 = mask.reshape(_BATCH // 2, _TOK)
    pooled = _pool(x2, m2, table)
    return _linear(pooled, W.T, b.reshape(1, _DIM))
